# baseline (speedup 1.0000x reference)
import jax
import jax.numpy as jnp
from jax import lax
from jax.experimental import pallas as pl
from jax.experimental.pallas import tpu as pltpu
from jax.experimental.pallas import tpu_sc as plsc

N_NODES = 10000
D_IN_PAD = 168
D_HID = 128
D_OUT_PAD = 8

NC = 2
NS = 16
NW = NC * NS
CHUNK = 128
E_PAD = 327680
CHUNKS_PER_TILE = E_PAD // (NW * CHUNK)
G_C0 = 80
G_C1 = 2 * CHUNKS_PER_TILE - G_C0
GROUP = 8
ACC_ROWS = 10240
RPT = ACC_ROWS // NS

_mesh = plsc.VectorSubcoreMesh(core_axis_name="c", subcore_axis_name="s")

BM = 400
GRID = N_NODES // BM


def _tc_lin1(x_ref, wl_ref, wr_ref, b1_ref, y1_ref, y1b_ref, xr_ref):
    xb = x_ref[...]
    y1 = jnp.dot(xb, wl_ref[...], preferred_element_type=jnp.float32)
    y1_ref[...] = y1
    y1b_ref[...] = y1
    xr_ref[...] = (
        jnp.dot(xb, wr_ref[...], preferred_element_type=jnp.float32) + b1_ref[...]
    )


def _tc_mid(a0_ref, a1_ref, c0_ref, c1_ref, xr_ref, wl_ref, wr_ref, b2_ref,
            y2_ref, hr_ref):
    cnt = jnp.maximum(c0_ref[...] + c1_ref[...], 1.0)
    mean = (a0_ref[...] + a1_ref[...]) / cnt
    h = jnp.maximum(mean + xr_ref[...], 0.0)
    y2_ref[...] = jnp.dot(h, wl_ref[...], preferred_element_type=jnp.float32)
    hr_ref[...] = (
        jnp.dot(h, wr_ref[...], preferred_element_type=jnp.float32) + b2_ref[...]
    )


def _tc_out(g00_ref, g01_ref, g10_ref, g11_ref, c0_ref, c1_ref, hr_ref, o_ref):
    cnt = jnp.maximum(c0_ref[...] + c1_ref[...], 1.0)
    z0 = (g00_ref[...] + g01_ref[...]) / cnt + hr_ref[...][:, 0:1]
    z1 = (g10_ref[...] + g11_ref[...]) / cnt + hr_ref[...][:, 1:2]
    m = jnp.maximum(z0, z1)
    lse = m + jnp.log(jnp.exp(z0 - m) + jnp.exp(z1 - m))
    o_ref[...] = jnp.concatenate([z0 - lse, z1 - lse], axis=1)


def _fill_vmem_2d(ref, nrows, value):
    v = jnp.full((16,), value, jnp.float32)

    def row(i, carry):
        for cg in range(8):
            ref[i, pl.ds(cg * 16, 16)] = v
        return carry

    lax.fori_loop(0, nrows, row, 0)


def _sc1_body(ya_hbm, yb_hbm, src_hbm, dst_hbm,
              acc_out, cnt_out,
              acc_sh, cnt_sh, src_v, dst_v, rows0_v, rows1_v, ones_v,
              semg0, semg1, sems0, sems1, semc):
    c = lax.axis_index("c")
    s = lax.axis_index("s")
    n_groups = lax.select(c == 0, G_C0 // GROUP, G_C1 // GROUP)
    tile_chunk0 = lax.select(c == 0, s * G_C0, NS * G_C0 + s * G_C1)
    _fill_vmem_2d(rows0_v, CHUNK, 0.0)
    one_v = jnp.full((16,), 1.0, jnp.float32)
    for cg in range(8):
        ones_v[pl.ds(cg * 16, 16)] = one_v
    for k in range(RPT // CHUNK):
        pltpu.sync_copy(rows0_v, acc_sh.at[pl.ds(s * RPT + k * CHUNK, CHUNK)])
        pltpu.sync_copy(rows0_v.at[0],
                        cnt_sh.at[pl.ds(s * RPT + k * CHUNK, CHUNK)])
    plsc.subcore_barrier()

    rows = (rows0_v, rows1_v)
    semg = (semg0, semg1)
    sems = (sems0, sems1)

    def run(y_hbm):
        def body(g, carry):
            base = tile_chunk0 + g * GROUP
            pltpu.sync_copy(src_hbm.at[pl.ds(base, GROUP)], src_v)
            pltpu.sync_copy(dst_hbm.at[pl.ds(base, GROUP)], dst_v)
            gat = [None, None]
            sca = [None, None]
            gat[0] = pltpu.async_copy(y_hbm.at[src_v.at[0]], rows[0], semg[0])
            cnt_cps = []
            for jj in range(GROUP):
                b = jj % 2
                nb = (jj + 1) % 2
                if jj + 1 < GROUP:
                    if sca[nb] is not None:
                        sca[nb].wait()
                        sca[nb] = None
                    gat[nb] = pltpu.async_copy(
                        y_hbm.at[src_v.at[jj + 1]], rows[nb], semg[nb])
                gat[b].wait()
                sca[b] = pltpu.async_copy(rows[b], acc_sh.at[dst_v.at[jj]],
                                          sems[b], add=True)
                cnt_cps.append(
                    pltpu.async_copy(ones_v, cnt_sh.at[dst_v.at[jj]], semc,
                                     add=True))
            for cp in sca:
                if cp is not None:
                    cp.wait()
            for cp in cnt_cps:
                cp.wait()
            return carry

        lax.fori_loop(0, n_groups, body, 0)

    @pl.when(c == 0)
    def _():
        run(ya_hbm)

    @pl.when(c != 0)
    def _():
        run(yb_hbm)

    plsc.subcore_barrier()
    pltpu.sync_copy(acc_sh.at[pl.ds(s * RPT, RPT)],
                    acc_out.at[c, pl.ds(s * RPT, RPT)])
    pltpu.sync_copy(cnt_sh.at[pl.ds(s * RPT, RPT)],
                    cnt_out.at[pl.ds(c * ACC_ROWS + s * RPT, RPT)])


_sc1 = pl.kernel(
    _sc1_body,
    out_type=[
        jax.ShapeDtypeStruct((NC, ACC_ROWS, D_HID), jnp.float32),
        jax.ShapeDtypeStruct((NC * ACC_ROWS,), jnp.float32),
    ],
    scratch_types=[
        pltpu.VMEM_SHARED((ACC_ROWS, D_HID), jnp.float32),
        pltpu.VMEM_SHARED((ACC_ROWS,), jnp.float32),
        pltpu.VMEM((GROUP, CHUNK), jnp.int32),
        pltpu.VMEM((GROUP, CHUNK), jnp.int32),
        pltpu.VMEM((CHUNK, D_HID), jnp.float32),
        pltpu.VMEM((CHUNK, D_HID), jnp.float32),
        pltpu.VMEM((CHUNK,), jnp.float32),
        pltpu.SemaphoreType.DMA,
        pltpu.SemaphoreType.DMA,
        pltpu.SemaphoreType.DMA,
        pltpu.SemaphoreType.DMA,
        pltpu.SemaphoreType.DMA,
    ],
    mesh=_mesh,
)


GROUP2 = 8


def _sc2_body(ya_hbm, yb_hbm, src_hbm, dst_hbm,
              acc_out,
              acca_sh, accb_sh, src_v, dst_v, rowa_v, rowb_v, semg, sems):
    c = lax.axis_index("c")
    s = lax.axis_index("s")
    n_groups = lax.select(c == 0, G_C0 // GROUP2, G_C1 // GROUP2)
    tile_chunk0 = lax.select(c == 0, s * G_C0, NS * G_C0 + s * G_C1)
    _fill_vmem_2d(rowa_v, GROUP2, 0.0)
    for k in range(RPT // CHUNK):
        pltpu.sync_copy(rowa_v.at[0],
                        acca_sh.at[pl.ds(s * RPT + k * CHUNK, CHUNK)])
        pltpu.sync_copy(rowa_v.at[0],
                        accb_sh.at[pl.ds(s * RPT + k * CHUNK, CHUNK)])
    plsc.subcore_barrier()

    def body(g, carry):
        base = tile_chunk0 + g * GROUP2
        pltpu.sync_copy(src_hbm.at[pl.ds(base, GROUP2)], src_v)
        pltpu.sync_copy(dst_hbm.at[pl.ds(base, GROUP2)], dst_v)
        gcps = []
        for jj in range(GROUP2):
            gcps.append(
                pltpu.async_copy(ya_hbm.at[src_v.at[jj]], rowa_v.at[jj], semg))
            gcps.append(
                pltpu.async_copy(yb_hbm.at[src_v.at[jj]], rowb_v.at[jj], semg))
        for cp in gcps:
            cp.wait()
        scps = []
        for jj in range(GROUP2):
            scps.append(
                pltpu.async_copy(rowa_v.at[jj], acca_sh.at[dst_v.at[jj]],
                                 sems, add=True))
            scps.append(
                pltpu.async_copy(rowb_v.at[jj], accb_sh.at[dst_v.at[jj]],
                                 sems, add=True))
        for cp in scps:
            cp.wait()
        return carry

    lax.fori_loop(0, n_groups, body, 0)
    plsc.subcore_barrier()
    pltpu.sync_copy(acca_sh.at[pl.ds(s * RPT, RPT)],
                    acc_out.at[0, pl.ds(c * ACC_ROWS + s * RPT, RPT)])
    pltpu.sync_copy(accb_sh.at[pl.ds(s * RPT, RPT)],
                    acc_out.at[1, pl.ds(c * ACC_ROWS + s * RPT, RPT)])


_sc2 = pl.kernel(
    _sc2_body,
    out_type=jax.ShapeDtypeStruct((2, NC * ACC_ROWS), jnp.float32),
    scratch_types=[
        pltpu.VMEM_SHARED((ACC_ROWS,), jnp.float32),
        pltpu.VMEM_SHARED((ACC_ROWS,), jnp.float32),
        pltpu.VMEM((GROUP2, CHUNK), jnp.int32),
        pltpu.VMEM((GROUP2, CHUNK), jnp.int32),
        pltpu.VMEM((GROUP2, CHUNK), jnp.float32),
        pltpu.VMEM((GROUP2, CHUNK), jnp.float32),
        pltpu.SemaphoreType.DMA,
        pltpu.SemaphoreType.DMA,
    ],
    mesh=_mesh,
)


def kernel(x, edge_index, W1_l, b1, W1_r, W2_l, b2, W2_r):
    n_edges = edge_index.shape[1]
    ei = edge_index.astype(jnp.int32)
    pad = E_PAD - n_edges
    src = jnp.concatenate([ei[0], jnp.zeros((pad,), jnp.int32)])
    pad_dst = N_NODES + (jnp.arange(pad, dtype=jnp.int32) % (ACC_ROWS - N_NODES))
    dst = jnp.concatenate([ei[1], pad_dst])
    src2d = src.reshape(E_PAD // CHUNK, CHUNK)
    dst2d = dst.reshape(E_PAD // CHUNK, CHUNK)

    xp = jnp.pad(x, ((0, 0), (0, D_IN_PAD - x.shape[1])))
    W1l_p = jnp.pad(W1_l, ((0, D_IN_PAD - W1_l.shape[0]), (0, 0)))
    W1r_p = jnp.pad(W1_r, ((0, D_IN_PAD - W1_r.shape[0]), (0, 0)))
    b1_2d = b1[None, :]
    W2l_p = jnp.pad(W2_l, ((0, 0), (0, D_OUT_PAD - W2_l.shape[1])))
    W2r_p = jnp.pad(W2_r, ((0, 0), (0, D_OUT_PAD - W2_r.shape[1])))
    b2_2d = jnp.pad(b2, (0, D_OUT_PAD - b2.shape[0]))[None, :]

    y1, y1b, xr = pl.pallas_call(
        _tc_lin1,
        grid=(GRID,),
        in_specs=[
            pl.BlockSpec((BM, D_IN_PAD), lambda i: (i, 0)),
            pl.BlockSpec((D_IN_PAD, D_HID), lambda i: (0, 0)),
            pl.BlockSpec((D_IN_PAD, D_HID), lambda i: (0, 0)),
            pl.BlockSpec((1, D_HID), lambda i: (0, 0)),
        ],
        out_specs=[
            pl.BlockSpec((BM, D_HID), lambda i: (i, 0)),
            pl.BlockSpec((BM, D_HID), lambda i: (i, 0)),
            pl.BlockSpec((BM, D_HID), lambda i: (i, 0)),
        ],
        out_shape=[jax.ShapeDtypeStruct((N_NODES, D_HID), jnp.float32)] * 3,
    )(xp, W1l_p, W1r_p, b1_2d)

    acc1, cnt = _sc1(y1, y1b, src2d, dst2d)
    a0 = acc1[0, :N_NODES]
    a1 = acc1[1, :N_NODES]
    c0 = cnt[:N_NODES][:, None]
    c1 = cnt[ACC_ROWS:ACC_ROWS + N_NODES][:, None]

    y2, hr = pl.pallas_call(
        _tc_mid,
        grid=(GRID,),
        in_specs=[
            pl.BlockSpec((BM, D_HID), lambda i: (i, 0)),
            pl.BlockSpec((BM, D_HID), lambda i: (i, 0)),
            pl.BlockSpec((BM, 1), lambda i: (i, 0)),
            pl.BlockSpec((BM, 1), lambda i: (i, 0)),
            pl.BlockSpec((BM, D_HID), lambda i: (i, 0)),
            pl.BlockSpec((D_HID, D_OUT_PAD), lambda i: (0, 0)),
            pl.BlockSpec((D_HID, D_OUT_PAD), lambda i: (0, 0)),
            pl.BlockSpec((1, D_OUT_PAD), lambda i: (0, 0)),
        ],
        out_specs=[
            pl.BlockSpec((BM, D_OUT_PAD), lambda i: (i, 0)),
            pl.BlockSpec((BM, D_OUT_PAD), lambda i: (i, 0)),
        ],
        out_shape=[jax.ShapeDtypeStruct((N_NODES, D_OUT_PAD), jnp.float32)] * 2,
    )(a0, a1, c0, c1, xr, W2l_p, W2r_p, b2_2d)

    y2a = y2[:, 0]
    y2b = y2[:, 1]
    acc2 = _sc2(y2a, y2b, src2d, dst2d)
    g00 = acc2[0, :N_NODES][:, None]
    g01 = acc2[0, ACC_ROWS:ACC_ROWS + N_NODES][:, None]
    g10 = acc2[1, :N_NODES][:, None]
    g11 = acc2[1, ACC_ROWS:ACC_ROWS + N_NODES][:, None]

    out = pl.pallas_call(
        _tc_out,
        grid=(GRID,),
        in_specs=[
            pl.BlockSpec((BM, 1), lambda i: (i, 0)),
            pl.BlockSpec((BM, 1), lambda i: (i, 0)),
            pl.BlockSpec((BM, 1), lambda i: (i, 0)),
            pl.BlockSpec((BM, 1), lambda i: (i, 0)),
            pl.BlockSpec((BM, 1), lambda i: (i, 0)),
            pl.BlockSpec((BM, 1), lambda i: (i, 0)),
            pl.BlockSpec((BM, D_OUT_PAD), lambda i: (i, 0)),
        ],
        out_specs=pl.BlockSpec((BM, 2), lambda i: (i, 0)),
        out_shape=jax.ShapeDtypeStruct((N_NODES, 2), jnp.float32),
    )(g00, g01, g10, g11, c0, c1, hr)

    return out


# speedup vs baseline: 1.0268x; 1.0268x over previous
import jax
import jax.numpy as jnp
from jax import lax
from jax.experimental import pallas as pl
from jax.experimental.pallas import tpu as pltpu
from jax.experimental.pallas import tpu_sc as plsc

N_NODES = 10000
D_IN_PAD = 168
D_HID = 128
D_OUT_PAD = 8

NC = 2
NS = 16
NW = NC * NS
CHUNK = 128
E_PAD = 327680
CHUNKS_PER_TILE = E_PAD // (NW * CHUNK)
G_C0 = 96
G_C1 = 2 * CHUNKS_PER_TILE - G_C0
GROUP = 8
ACC_ROWS = 10240
RPT = ACC_ROWS // NS

_mesh = plsc.VectorSubcoreMesh(core_axis_name="c", subcore_axis_name="s")

BM = 400
GRID = N_NODES // BM


def _tc_lin1(x_ref, wl_ref, wr_ref, b1_ref, y1_ref, xr_ref):
    xb = x_ref[...]
    y1_ref[...] = jnp.dot(xb, wl_ref[...], preferred_element_type=jnp.float32)
    xr_ref[...] = (
        jnp.dot(xb, wr_ref[...], preferred_element_type=jnp.float32) + b1_ref[...]
    )


def _tc_mid(a0_ref, c0_ref, xr_ref, wl_ref, wr_ref, b2_ref,
            y2_ref, hr_ref):
    cnt = jnp.maximum(c0_ref[...], 1.0)
    mean = a0_ref[...] / cnt
    h = jnp.maximum(mean + xr_ref[...], 0.0)
    y2_ref[...] = jnp.dot(h, wl_ref[...], preferred_element_type=jnp.float32)
    hr_ref[...] = (
        jnp.dot(h, wr_ref[...], preferred_element_type=jnp.float32) + b2_ref[...]
    )


def _tc_out(g00_ref, g01_ref, g10_ref, g11_ref, c0_ref, hr_ref, o_ref):
    cnt = jnp.maximum(c0_ref[...], 1.0)
    z0 = (g00_ref[...] + g01_ref[...]) / cnt + hr_ref[...][:, 0:1]
    z1 = (g10_ref[...] + g11_ref[...]) / cnt + hr_ref[...][:, 1:2]
    m = jnp.maximum(z0, z1)
    lse = m + jnp.log(jnp.exp(z0 - m) + jnp.exp(z1 - m))
    o_ref[...] = jnp.concatenate([z0 - lse, z1 - lse], axis=1)


def _fill_vmem_2d(ref, nrows, value):
    v = jnp.full((16,), value, jnp.float32)

    def row(i, carry):
        for cg in range(8):
            ref[i, pl.ds(cg * 16, 16)] = v
        return carry

    lax.fori_loop(0, nrows, row, 0)


G_ALL = 2 * CHUNKS_PER_TILE


def _sc1_body(y_hbm, src_hbm, dst_hbm,
              acc_out, cnt_out,
              acc_sh, cnt_sh, src_v, dst_v, rows0_v, rows1_v, ones_v,
              semg0, semg1, sems0, sems1, semc):
    c = lax.axis_index("c")
    s = lax.axis_index("s")
    _fill_vmem_2d(rows0_v, CHUNK, 0.0)
    one_v = jnp.full((16,), 1.0, jnp.float32)
    for cg in range(8):
        ones_v[pl.ds(cg * 16, 16)] = one_v
    for k in range(RPT // CHUNK):
        pltpu.sync_copy(rows0_v, acc_sh.at[pl.ds(s * RPT + k * CHUNK, CHUNK)])
        pltpu.sync_copy(rows0_v.at[0],
                        cnt_sh.at[pl.ds(s * RPT + k * CHUNK, CHUNK)])
    plsc.subcore_barrier()

    rows = (rows0_v, rows1_v)
    semg = (semg0, semg1)
    sems = (sems0, sems1)

    @pl.when(c == 0)
    def _():
        def body(g, carry):
            base = s * G_ALL + g * GROUP
            pltpu.sync_copy(src_hbm.at[pl.ds(base, GROUP)], src_v)
            pltpu.sync_copy(dst_hbm.at[pl.ds(base, GROUP)], dst_v)
            gat = [None, None]
            sca = [None, None]
            gat[0] = pltpu.async_copy(y_hbm.at[src_v.at[0]], rows[0], semg[0])
            for jj in range(GROUP):
                b = jj % 2
                nb = (jj + 1) % 2
                if jj + 1 < GROUP:
                    if sca[nb] is not None:
                        sca[nb].wait()
                        sca[nb] = None
                    gat[nb] = pltpu.async_copy(
                        y_hbm.at[src_v.at[jj + 1]], rows[nb], semg[nb])
                gat[b].wait()
                sca[b] = pltpu.async_copy(rows[b], acc_sh.at[dst_v.at[jj]],
                                          sems[b], add=True)
            for cp in sca:
                if cp is not None:
                    cp.wait()
            return carry

        lax.fori_loop(0, G_ALL // GROUP, body, 0)

    @pl.when(c != 0)
    def _():
        def body(g, carry):
            base = s * G_ALL + g * GROUP
            pltpu.sync_copy(dst_hbm.at[pl.ds(base, GROUP)], dst_v)
            cnt_cps = []
            for jj in range(GROUP):
                cnt_cps.append(
                    pltpu.async_copy(ones_v, cnt_sh.at[dst_v.at[jj]], semc,
                                     add=True))
            for cp in cnt_cps:
                cp.wait()
            return carry

        lax.fori_loop(0, G_ALL // GROUP, body, 0)

    plsc.subcore_barrier()

    @pl.when(c == 0)
    def _():
        pltpu.sync_copy(acc_sh.at[pl.ds(s * RPT, RPT)],
                        acc_out.at[pl.ds(s * RPT, RPT)])

    @pl.when(c != 0)
    def _():
        pltpu.sync_copy(cnt_sh.at[pl.ds(s * RPT, RPT)],
                        cnt_out.at[pl.ds(s * RPT, RPT)])


_sc1 = pl.kernel(
    _sc1_body,
    out_type=[
        jax.ShapeDtypeStruct((ACC_ROWS, D_HID), jnp.float32),
        jax.ShapeDtypeStruct((ACC_ROWS,), jnp.float32),
    ],
    scratch_types=[
        pltpu.VMEM_SHARED((ACC_ROWS, D_HID), jnp.float32),
        pltpu.VMEM_SHARED((ACC_ROWS,), jnp.float32),
        pltpu.VMEM((GROUP, CHUNK), jnp.int32),
        pltpu.VMEM((GROUP, CHUNK), jnp.int32),
        pltpu.VMEM((CHUNK, D_HID), jnp.float32),
        pltpu.VMEM((CHUNK, D_HID), jnp.float32),
        pltpu.VMEM((CHUNK,), jnp.float32),
        pltpu.SemaphoreType.DMA,
        pltpu.SemaphoreType.DMA,
        pltpu.SemaphoreType.DMA,
        pltpu.SemaphoreType.DMA,
        pltpu.SemaphoreType.DMA,
    ],
    mesh=_mesh,
)


GROUP2 = 8


def _sc2_body(ya_hbm, yb_hbm, src_hbm, dst_hbm,
              acc_out,
              acca_sh, accb_sh, src_v, dst_v, rowa_v, rowb_v, semg, sems):
    c = lax.axis_index("c")
    s = lax.axis_index("s")
    n_groups = lax.select(c == 0, G_C0 // GROUP2, G_C1 // GROUP2)
    tile_chunk0 = lax.select(c == 0, s * G_C0, NS * G_C0 + s * G_C1)
    _fill_vmem_2d(rowa_v, GROUP2, 0.0)
    for k in range(RPT // CHUNK):
        pltpu.sync_copy(rowa_v.at[0],
                        acca_sh.at[pl.ds(s * RPT + k * CHUNK, CHUNK)])
        pltpu.sync_copy(rowa_v.at[0],
                        accb_sh.at[pl.ds(s * RPT + k * CHUNK, CHUNK)])
    plsc.subcore_barrier()

    def body(g, carry):
        base = tile_chunk0 + g * GROUP2
        pltpu.sync_copy(src_hbm.at[pl.ds(base, GROUP2)], src_v)
        pltpu.sync_copy(dst_hbm.at[pl.ds(base, GROUP2)], dst_v)
        gcps = []
        for jj in range(GROUP2):
            gcps.append(
                pltpu.async_copy(ya_hbm.at[src_v.at[jj]], rowa_v.at[jj], semg))
            gcps.append(
                pltpu.async_copy(yb_hbm.at[src_v.at[jj]], rowb_v.at[jj], semg))
        for cp in gcps:
            cp.wait()
        scps = []
        for jj in range(GROUP2):
            scps.append(
                pltpu.async_copy(rowa_v.at[jj], acca_sh.at[dst_v.at[jj]],
                                 sems, add=True))
            scps.append(
                pltpu.async_copy(rowb_v.at[jj], accb_sh.at[dst_v.at[jj]],
                                 sems, add=True))
        for cp in scps:
            cp.wait()
        return carry

    lax.fori_loop(0, n_groups, body, 0)
    plsc.subcore_barrier()
    pltpu.sync_copy(acca_sh.at[pl.ds(s * RPT, RPT)],
                    acc_out.at[0, pl.ds(c * ACC_ROWS + s * RPT, RPT)])
    pltpu.sync_copy(accb_sh.at[pl.ds(s * RPT, RPT)],
                    acc_out.at[1, pl.ds(c * ACC_ROWS + s * RPT, RPT)])


_sc2 = pl.kernel(
    _sc2_body,
    out_type=jax.ShapeDtypeStruct((2, NC * ACC_ROWS), jnp.float32),
    scratch_types=[
        pltpu.VMEM_SHARED((ACC_ROWS,), jnp.float32),
        pltpu.VMEM_SHARED((ACC_ROWS,), jnp.float32),
        pltpu.VMEM((GROUP2, CHUNK), jnp.int32),
        pltpu.VMEM((GROUP2, CHUNK), jnp.int32),
        pltpu.VMEM((GROUP2, CHUNK), jnp.float32),
        pltpu.VMEM((GROUP2, CHUNK), jnp.float32),
        pltpu.SemaphoreType.DMA,
        pltpu.SemaphoreType.DMA,
    ],
    mesh=_mesh,
)


def kernel(x, edge_index, W1_l, b1, W1_r, W2_l, b2, W2_r):
    n_edges = edge_index.shape[1]
    ei = edge_index.astype(jnp.int32)
    pad = E_PAD - n_edges
    src = jnp.concatenate([ei[0], jnp.zeros((pad,), jnp.int32)])
    pad_dst = N_NODES + (jnp.arange(pad, dtype=jnp.int32) % (ACC_ROWS - N_NODES))
    dst = jnp.concatenate([ei[1], pad_dst])
    src2d = src.reshape(E_PAD // CHUNK, CHUNK)
    dst2d = dst.reshape(E_PAD // CHUNK, CHUNK)

    xp = jnp.pad(x, ((0, 0), (0, D_IN_PAD - x.shape[1])))
    W1l_p = jnp.pad(W1_l, ((0, D_IN_PAD - W1_l.shape[0]), (0, 0)))
    W1r_p = jnp.pad(W1_r, ((0, D_IN_PAD - W1_r.shape[0]), (0, 0)))
    b1_2d = b1[None, :]
    W2l_p = jnp.pad(W2_l, ((0, 0), (0, D_OUT_PAD - W2_l.shape[1])))
    W2r_p = jnp.pad(W2_r, ((0, 0), (0, D_OUT_PAD - W2_r.shape[1])))
    b2_2d = jnp.pad(b2, (0, D_OUT_PAD - b2.shape[0]))[None, :]

    y1, xr = pl.pallas_call(
        _tc_lin1,
        grid=(GRID,),
        in_specs=[
            pl.BlockSpec((BM, D_IN_PAD), lambda i: (i, 0)),
            pl.BlockSpec((D_IN_PAD, D_HID), lambda i: (0, 0)),
            pl.BlockSpec((D_IN_PAD, D_HID), lambda i: (0, 0)),
            pl.BlockSpec((1, D_HID), lambda i: (0, 0)),
        ],
        out_specs=[
            pl.BlockSpec((BM, D_HID), lambda i: (i, 0)),
            pl.BlockSpec((BM, D_HID), lambda i: (i, 0)),
        ],
        out_shape=[jax.ShapeDtypeStruct((N_NODES, D_HID), jnp.float32)] * 2,
    )(xp, W1l_p, W1r_p, b1_2d)

    acc1, cnt = _sc1(y1, src2d, dst2d)
    a0 = acc1[:N_NODES]
    c0 = cnt[:N_NODES][:, None]

    y2, hr = pl.pallas_call(
        _tc_mid,
        grid=(GRID,),
        in_specs=[
            pl.BlockSpec((BM, D_HID), lambda i: (i, 0)),
            pl.BlockSpec((BM, 1), lambda i: (i, 0)),
            pl.BlockSpec((BM, D_HID), lambda i: (i, 0)),
            pl.BlockSpec((D_HID, D_OUT_PAD), lambda i: (0, 0)),
            pl.BlockSpec((D_HID, D_OUT_PAD), lambda i: (0, 0)),
            pl.BlockSpec((1, D_OUT_PAD), lambda i: (0, 0)),
        ],
        out_specs=[
            pl.BlockSpec((BM, D_OUT_PAD), lambda i: (i, 0)),
            pl.BlockSpec((BM, D_OUT_PAD), lambda i: (i, 0)),
        ],
        out_shape=[jax.ShapeDtypeStruct((N_NODES, D_OUT_PAD), jnp.float32)] * 2,
    )(a0, c0, xr, W2l_p, W2r_p, b2_2d)

    y2a = y2[:, 0]
    y2b = y2[:, 1]
    acc2 = _sc2(y2a, y2b, src2d, dst2d)
    g00 = acc2[0, :N_NODES][:, None]
    g01 = acc2[0, ACC_ROWS:ACC_ROWS + N_NODES][:, None]
    g10 = acc2[1, :N_NODES][:, None]
    g11 = acc2[1, ACC_ROWS:ACC_ROWS + N_NODES][:, None]

    out = pl.pallas_call(
        _tc_out,
        grid=(GRID,),
        in_specs=[
            pl.BlockSpec((BM, 1), lambda i: (i, 0)),
            pl.BlockSpec((BM, 1), lambda i: (i, 0)),
            pl.BlockSpec((BM, 1), lambda i: (i, 0)),
            pl.BlockSpec((BM, 1), lambda i: (i, 0)),
            pl.BlockSpec((BM, 1), lambda i: (i, 0)),
            pl.BlockSpec((BM, D_OUT_PAD), lambda i: (i, 0)),
        ],
        out_specs=pl.BlockSpec((BM, 2), lambda i: (i, 0)),
        out_shape=jax.ShapeDtypeStruct((N_NODES, 2), jnp.float32),
    )(g00, g01, g10, g11, c0, hr)

    return out


# speedup vs baseline: 1.2435x; 1.2111x over previous
import jax
import jax.numpy as jnp
from jax import lax
from jax.experimental import pallas as pl
from jax.experimental.pallas import tpu as pltpu
from jax.experimental.pallas import tpu_sc as plsc

N_NODES = 10000
D_IN_PAD = 168
D_HID = 128
D_OUT_PAD = 8

NC = 2
NS = 16
NW = NC * NS
CHUNK = 128
E_PAD = 327680
CHUNKS_PER_TILE = E_PAD // (NW * CHUNK)
G_C0 = 120
G_C1 = 2 * CHUNKS_PER_TILE - G_C0
GROUP = 8
ACC_ROWS = 10240
RPT = ACC_ROWS // NS

_mesh = plsc.VectorSubcoreMesh(core_axis_name="c", subcore_axis_name="s")

BM = 400
GRID = N_NODES // BM


def _tc_lin1(x_ref, wl_ref, wr_ref, b1_ref, y1_ref, xr_ref):
    xb = x_ref[...]
    y1_ref[...] = jnp.dot(xb, wl_ref[...], preferred_element_type=jnp.float32)
    xr_ref[...] = (
        jnp.dot(xb, wr_ref[...], preferred_element_type=jnp.float32) + b1_ref[...]
    )


def _tc_mid(a0_ref, a1_ref, c0_ref, c1_ref, xr_ref, wl_ref, wr_ref, b2_ref,
            y2_ref, hr_ref):
    cnt = jnp.maximum(c0_ref[...] + c1_ref[...], 1.0)
    mean = (a0_ref[...] + a1_ref[...]) / cnt
    h = jnp.maximum(mean + xr_ref[...], 0.0)
    y2_ref[...] = jnp.dot(h, wl_ref[...], preferred_element_type=jnp.float32)
    hr_ref[...] = (
        jnp.dot(h, wr_ref[...], preferred_element_type=jnp.float32) + b2_ref[...]
    )


def _tc_out(g00_ref, g01_ref, g10_ref, g11_ref, c0_ref, c1_ref, hr_ref, o_ref):
    cnt = jnp.maximum(c0_ref[...] + c1_ref[...], 1.0)
    z0 = (g00_ref[...] + g01_ref[...]) / cnt + hr_ref[...][:, 0:1]
    z1 = (g10_ref[...] + g11_ref[...]) / cnt + hr_ref[...][:, 1:2]
    m = jnp.maximum(z0, z1)
    lse = m + jnp.log(jnp.exp(z0 - m) + jnp.exp(z1 - m))
    o_ref[...] = jnp.concatenate([z0 - lse, z1 - lse], axis=1)


def _fill_vmem_2d(ref, nrows, value):
    v = jnp.full((16,), value, jnp.float32)

    def row(i, carry):
        for cg in range(8):
            ref[i, pl.ds(cg * 16, 16)] = v
        return carry

    lax.fori_loop(0, nrows, row, 0)


def _sc1_body(y_hbm, src_hbm, dst_hbm,
              acc_out, cnt_out,
              acc_sh, cnt_sh, src_v, dst_v, rows0_v, rows1_v, ones_v,
              semg0, semg1, sems0, sems1, semc):
    c = lax.axis_index("c")
    s = lax.axis_index("s")
    n_groups = lax.select(c == 0, G_C0 // GROUP, G_C1 // GROUP)
    tile_chunk0 = lax.select(c == 0, s * G_C0, NS * G_C0 + s * G_C1)
    _fill_vmem_2d(rows0_v, CHUNK, 0.0)
    one_v = jnp.full((16,), 1.0, jnp.float32)
    for cg in range(8):
        ones_v[pl.ds(cg * 16, 16)] = one_v
    for k in range(RPT // CHUNK):
        pltpu.sync_copy(rows0_v, acc_sh.at[pl.ds(s * RPT + k * CHUNK, CHUNK)])
        pltpu.sync_copy(rows0_v.at[0],
                        cnt_sh.at[pl.ds(s * RPT + k * CHUNK, CHUNK)])
    plsc.subcore_barrier()

    rows = (rows0_v, rows1_v)
    semg = (semg0, semg1)
    sems = (sems0, sems1)

    def body(g, carry):
        base = tile_chunk0 + g * GROUP
        pltpu.sync_copy(src_hbm.at[pl.ds(base, GROUP)], src_v)
        pltpu.sync_copy(dst_hbm.at[pl.ds(base, GROUP)], dst_v)
        gat = [None, None]
        sca = [None, None]
        gat[0] = pltpu.async_copy(y_hbm.at[src_v.at[0]], rows[0], semg[0])
        cnt_cps = []
        for jj in range(GROUP):
            b = jj % 2
            nb = (jj + 1) % 2
            if jj + 1 < GROUP:
                if sca[nb] is not None:
                    sca[nb].wait()
                    sca[nb] = None
                gat[nb] = pltpu.async_copy(
                    y_hbm.at[src_v.at[jj + 1]], rows[nb], semg[nb])
            gat[b].wait()
            sca[b] = pltpu.async_copy(rows[b], acc_sh.at[dst_v.at[jj]],
                                      sems[b], add=True)
            cnt_cps.append(
                pltpu.async_copy(ones_v, cnt_sh.at[dst_v.at[jj]], semc,
                                 add=True))
        for cp in sca:
            if cp is not None:
                cp.wait()
        for cp in cnt_cps:
            cp.wait()
        return carry

    lax.fori_loop(0, n_groups, body, 0)
    plsc.subcore_barrier()
    pltpu.sync_copy(acc_sh.at[pl.ds(s * RPT, RPT)],
                    acc_out.at[c, pl.ds(s * RPT, RPT)])
    pltpu.sync_copy(cnt_sh.at[pl.ds(s * RPT, RPT)],
                    cnt_out.at[pl.ds(c * ACC_ROWS + s * RPT, RPT)])


_sc1 = pl.kernel(
    _sc1_body,
    out_type=[
        jax.ShapeDtypeStruct((NC, ACC_ROWS, D_HID), jnp.float32),
        jax.ShapeDtypeStruct((NC * ACC_ROWS,), jnp.float32),
    ],
    scratch_types=[
        pltpu.VMEM_SHARED((ACC_ROWS, D_HID), jnp.float32),
        pltpu.VMEM_SHARED((ACC_ROWS,), jnp.float32),
        pltpu.VMEM((GROUP, CHUNK), jnp.int32),
        pltpu.VMEM((GROUP, CHUNK), jnp.int32),
        pltpu.VMEM((CHUNK, D_HID), jnp.float32),
        pltpu.VMEM((CHUNK, D_HID), jnp.float32),
        pltpu.VMEM((CHUNK,), jnp.float32),
        pltpu.SemaphoreType.DMA,
        pltpu.SemaphoreType.DMA,
        pltpu.SemaphoreType.DMA,
        pltpu.SemaphoreType.DMA,
        pltpu.SemaphoreType.DMA,
    ],
    mesh=_mesh,
)


GROUP2 = 8


def _sc2_body(ya_hbm, yb_hbm, src_hbm, dst_hbm,
              acc_out,
              acca_sh, accb_sh, src_v, dst_v, rowa_v, rowb_v, semg, sems):
    c = lax.axis_index("c")
    s = lax.axis_index("s")
    n_groups = lax.select(c == 0, G_C0 // GROUP2, G_C1 // GROUP2)
    tile_chunk0 = lax.select(c == 0, s * G_C0, NS * G_C0 + s * G_C1)
    _fill_vmem_2d(rowa_v, GROUP2, 0.0)
    for k in range(RPT // CHUNK):
        pltpu.sync_copy(rowa_v.at[0],
                        acca_sh.at[pl.ds(s * RPT + k * CHUNK, CHUNK)])
        pltpu.sync_copy(rowa_v.at[0],
                        accb_sh.at[pl.ds(s * RPT + k * CHUNK, CHUNK)])
    plsc.subcore_barrier()

    def body(g, carry):
        base = tile_chunk0 + g * GROUP2
        pltpu.sync_copy(src_hbm.at[pl.ds(base, GROUP2)], src_v)
        pltpu.sync_copy(dst_hbm.at[pl.ds(base, GROUP2)], dst_v)
        gcps = []
        for jj in range(GROUP2):
            gcps.append(
                pltpu.async_copy(ya_hbm.at[src_v.at[jj]], rowa_v.at[jj], semg))
            gcps.append(
                pltpu.async_copy(yb_hbm.at[src_v.at[jj]], rowb_v.at[jj], semg))
        for cp in gcps:
            cp.wait()
        scps = []
        for jj in range(GROUP2):
            scps.append(
                pltpu.async_copy(rowa_v.at[jj], acca_sh.at[dst_v.at[jj]],
                                 sems, add=True))
            scps.append(
                pltpu.async_copy(rowb_v.at[jj], accb_sh.at[dst_v.at[jj]],
                                 sems, add=True))
        for cp in scps:
            cp.wait()
        return carry

    lax.fori_loop(0, n_groups, body, 0)
    plsc.subcore_barrier()
    pltpu.sync_copy(acca_sh.at[pl.ds(s * RPT, RPT)],
                    acc_out.at[0, pl.ds(c * ACC_ROWS + s * RPT, RPT)])
    pltpu.sync_copy(accb_sh.at[pl.ds(s * RPT, RPT)],
                    acc_out.at[1, pl.ds(c * ACC_ROWS + s * RPT, RPT)])


_sc2 = pl.kernel(
    _sc2_body,
    out_type=jax.ShapeDtypeStruct((2, NC * ACC_ROWS), jnp.float32),
    scratch_types=[
        pltpu.VMEM_SHARED((ACC_ROWS,), jnp.float32),
        pltpu.VMEM_SHARED((ACC_ROWS,), jnp.float32),
        pltpu.VMEM((GROUP2, CHUNK), jnp.int32),
        pltpu.VMEM((GROUP2, CHUNK), jnp.int32),
        pltpu.VMEM((GROUP2, CHUNK), jnp.float32),
        pltpu.VMEM((GROUP2, CHUNK), jnp.float32),
        pltpu.SemaphoreType.DMA,
        pltpu.SemaphoreType.DMA,
    ],
    mesh=_mesh,
)


def kernel(x, edge_index, W1_l, b1, W1_r, W2_l, b2, W2_r):
    n_edges = edge_index.shape[1]
    ei = edge_index.astype(jnp.int32)
    pad = E_PAD - n_edges
    src = jnp.concatenate([ei[0], jnp.zeros((pad,), jnp.int32)])
    pad_dst = N_NODES + (jnp.arange(pad, dtype=jnp.int32) % (ACC_ROWS - N_NODES))
    dst = jnp.concatenate([ei[1], pad_dst])
    src2d = src.reshape(E_PAD // CHUNK, CHUNK)
    dst2d = dst.reshape(E_PAD // CHUNK, CHUNK)

    xp = jnp.pad(x, ((0, 0), (0, D_IN_PAD - x.shape[1])))
    W1l_p = jnp.pad(W1_l, ((0, D_IN_PAD - W1_l.shape[0]), (0, 0)))
    W1r_p = jnp.pad(W1_r, ((0, D_IN_PAD - W1_r.shape[0]), (0, 0)))
    b1_2d = b1[None, :]
    W2l_p = jnp.pad(W2_l, ((0, 0), (0, D_OUT_PAD - W2_l.shape[1])))
    W2r_p = jnp.pad(W2_r, ((0, 0), (0, D_OUT_PAD - W2_r.shape[1])))
    b2_2d = jnp.pad(b2, (0, D_OUT_PAD - b2.shape[0]))[None, :]

    y1, xr = pl.pallas_call(
        _tc_lin1,
        grid=(GRID,),
        in_specs=[
            pl.BlockSpec((BM, D_IN_PAD), lambda i: (i, 0)),
            pl.BlockSpec((D_IN_PAD, D_HID), lambda i: (0, 0)),
            pl.BlockSpec((D_IN_PAD, D_HID), lambda i: (0, 0)),
            pl.BlockSpec((1, D_HID), lambda i: (0, 0)),
        ],
        out_specs=[
            pl.BlockSpec((BM, D_HID), lambda i: (i, 0)),
            pl.BlockSpec((BM, D_HID), lambda i: (i, 0)),
        ],
        out_shape=[jax.ShapeDtypeStruct((N_NODES, D_HID), jnp.float32)] * 2,
    )(xp, W1l_p, W1r_p, b1_2d)

    acc1, cnt = _sc1(y1, src2d, dst2d)
    a0 = acc1[0, :N_NODES]
    a1 = acc1[1, :N_NODES]
    c0 = cnt[:N_NODES][:, None]
    c1 = cnt[ACC_ROWS:ACC_ROWS + N_NODES][:, None]

    y2, hr = pl.pallas_call(
        _tc_mid,
        grid=(GRID,),
        in_specs=[
            pl.BlockSpec((BM, D_HID), lambda i: (i, 0)),
            pl.BlockSpec((BM, D_HID), lambda i: (i, 0)),
            pl.BlockSpec((BM, 1), lambda i: (i, 0)),
            pl.BlockSpec((BM, 1), lambda i: (i, 0)),
            pl.BlockSpec((BM, D_HID), lambda i: (i, 0)),
            pl.BlockSpec((D_HID, D_OUT_PAD), lambda i: (0, 0)),
            pl.BlockSpec((D_HID, D_OUT_PAD), lambda i: (0, 0)),
            pl.BlockSpec((1, D_OUT_PAD), lambda i: (0, 0)),
        ],
        out_specs=[
            pl.BlockSpec((BM, D_OUT_PAD), lambda i: (i, 0)),
            pl.BlockSpec((BM, D_OUT_PAD), lambda i: (i, 0)),
        ],
        out_shape=[jax.ShapeDtypeStruct((N_NODES, D_OUT_PAD), jnp.float32)] * 2,
    )(a0, a1, c0, c1, xr, W2l_p, W2r_p, b2_2d)

    y2a = y2[:, 0]
    y2b = y2[:, 1]
    acc2 = _sc2(y2a, y2b, src2d, dst2d)
    g00 = acc2[0, :N_NODES][:, None]
    g01 = acc2[0, ACC_ROWS:ACC_ROWS + N_NODES][:, None]
    g10 = acc2[1, :N_NODES][:, None]
    g11 = acc2[1, ACC_ROWS:ACC_ROWS + N_NODES][:, None]

    out = pl.pallas_call(
        _tc_out,
        grid=(GRID,),
        in_specs=[
            pl.BlockSpec((BM, 1), lambda i: (i, 0)),
            pl.BlockSpec((BM, 1), lambda i: (i, 0)),
            pl.BlockSpec((BM, 1), lambda i: (i, 0)),
            pl.BlockSpec((BM, 1), lambda i: (i, 0)),
            pl.BlockSpec((BM, 1), lambda i: (i, 0)),
            pl.BlockSpec((BM, 1), lambda i: (i, 0)),
            pl.BlockSpec((BM, D_OUT_PAD), lambda i: (i, 0)),
        ],
        out_specs=pl.BlockSpec((BM, 2), lambda i: (i, 0)),
        out_shape=jax.ShapeDtypeStruct((N_NODES, 2), jnp.float32),
    )(g00, g01, g10, g11, c0, c1, hr)

    return out
